# Initial kernel scaffold; baseline (speedup 1.0000x reference)
import jax
import jax.numpy as jnp
from jax.experimental import pallas as pl


def kernel(x, W, H, lin1_w, lin1_b, lin2_w, user_bias, item_bias):
    raise NotImplementedError("write your pallas kernel here")


# speedup vs baseline: 1.5651x; 1.5651x over previous
import functools

import jax
import jax.numpy as jnp
from jax import lax
from jax.experimental import pallas as pl
from jax.experimental.pallas import tpu as pltpu
from jax.experimental.pallas import tpu_sc as plsc

BATCH = 16384
EMB_K = 16

_NC, _NS = 2, 16
_NW = _NC * _NS
_BPW = BATCH // _NW
_CHB = 128
_NCH = _BPW // _CHB
_L = 16


@functools.cache
def _make_sc_gather():
    mesh = plsc.VectorSubcoreMesh(core_axis_name="c", subcore_axis_name="s")

    @functools.partial(
        pl.kernel,
        mesh=mesh,
        compiler_params=pltpu.CompilerParams(use_tc_tiling_on_sc=False,
                                             needs_layout_passes=False),
        out_type=[
            jax.ShapeDtypeStruct((BATCH, EMB_K), jnp.float32),
            jax.ShapeDtypeStruct((BATCH, EMB_K), jnp.float32),
            jax.ShapeDtypeStruct((BATCH,), jnp.float32),
        ],
        scratch_types=[
            pltpu.VMEM((_NCH, _CHB), jnp.int32),
            pltpu.VMEM((_NCH, _CHB), jnp.int32),
            pltpu.VMEM((_NCH, _CHB), jnp.int32),
            pltpu.VMEM((_NCH, _CHB), jnp.int32),
            pltpu.VMEM((_BPW, EMB_K), jnp.float32),
            pltpu.VMEM((_BPW, EMB_K), jnp.float32),
            pltpu.VMEM((_BPW, _L), jnp.float32),
            pltpu.VMEM((_BPW, _L), jnp.float32),
            pltpu.VMEM((_BPW,), jnp.float32),
            pltpu.SemaphoreType.DMA,
        ],
    )
    def gather_kernel(uidx_hbm, iidx_hbm, w_hbm, h_hbm, ub_hbm, ib_hbm,
                      uz_out, vz_out, bsum_out,
                      uidx_v, iidx_v, uhi_v, ihi_v,
                      urows_v, vrows_v, ubr_v, ibr_v, bsum_v, sem):
        wid = lax.axis_index("s") * _NC + lax.axis_index("c")
        base = wid * _BPW

        idx_cps = []
        for j in range(_NCH):
            idx_cps.append(pltpu.async_copy(
                uidx_hbm.at[pl.ds(base + j * _CHB, _CHB)], uidx_v.at[j], sem))
            idx_cps.append(pltpu.async_copy(
                iidx_hbm.at[pl.ds(base + j * _CHB, _CHB)], iidx_v.at[j], sem))
        for cp in idx_cps:
            cp.wait()

        cps = []
        for j in range(_NCH):
            r = pl.ds(j * _CHB, _CHB)
            cps.append(pltpu.async_copy(w_hbm.at[uidx_v.at[j]], urows_v.at[r], sem))
            cps.append(pltpu.async_copy(h_hbm.at[iidx_v.at[j]], vrows_v.at[r], sem))

        for j in range(_NCH):
            for g in range(_CHB // _L):
                s = pl.ds(g * _L, _L)
                uhi_v[j, s] = lax.shift_right_logical(uidx_v[j, s], 4)
                ihi_v[j, s] = lax.shift_right_logical(iidx_v[j, s], 4)

        for j in range(_NCH):
            r = pl.ds(j * _CHB, _CHB)
            cps.append(pltpu.async_copy(ub_hbm.at[uhi_v.at[j]], ubr_v.at[r], sem))
            cps.append(pltpu.async_copy(ib_hbm.at[ihi_v.at[j]], ibr_v.at[r], sem))
        for cp in cps:
            cp.wait()

        lane = lax.iota(jnp.int32, _L)
        for j in range(_NCH):
            for g in range(_CHB // _L):
                s = pl.ds(g * _L, _L)
                rows = jnp.full((_L,), j * _CHB + g * _L, jnp.int32) + lane
                ub_e = plsc.load_gather(ubr_v, [rows, uidx_v[j, s] & 15])
                ib_e = plsc.load_gather(ibr_v, [rows, iidx_v[j, s] & 15])
                bsum_v[pl.ds(j * _CHB + g * _L, _L)] = ub_e + ib_e

        out_cps = [
            pltpu.async_copy(urows_v, uz_out.at[pl.ds(base, _BPW)], sem),
            pltpu.async_copy(vrows_v, vz_out.at[pl.ds(base, _BPW)], sem),
            pltpu.async_copy(bsum_v, bsum_out.at[pl.ds(base, _BPW)], sem),
        ]
        for cp in out_cps:
            cp.wait()

    return gather_kernel


_BLK = 2048


def _mlp_body(uz_ref, vz_ref, bsum_ref, w1_ref, b1_ref, w2_ref, out_ref):
    uz = uz_ref[...]
    vz = vz_ref[...]
    w1 = w1_ref[...]
    h = lax.dot_general(uz, w1[:, :EMB_K], (((1,), (1,)), ((), ())),
                        preferred_element_type=jnp.float32)
    h = h + lax.dot_general(vz, w1[:, EMB_K:], (((1,), (1,)), ((), ())),
                            preferred_element_type=jnp.float32)
    h = jnp.maximum(h + b1_ref[...], 0.0)
    out = jnp.sum(h * w2_ref[...], axis=1, keepdims=True)
    out_ref[...] = out + bsum_ref[...]


def _mlp(uz, vz, bsum, w1, b1, w2):
    grid = (BATCH // _BLK,)
    row_blk = lambda i: (i, 0)
    w_blk = lambda i: (0, 0)
    return pl.pallas_call(
        _mlp_body,
        grid=grid,
        in_specs=[
            pl.BlockSpec((_BLK, EMB_K), row_blk),
            pl.BlockSpec((_BLK, EMB_K), row_blk),
            pl.BlockSpec((_BLK, 1), row_blk),
            pl.BlockSpec((EMB_K, 2 * EMB_K), w_blk),
            pl.BlockSpec((1, EMB_K), w_blk),
            pl.BlockSpec((1, EMB_K), w_blk),
        ],
        out_specs=pl.BlockSpec((_BLK, 1), row_blk),
        out_shape=jax.ShapeDtypeStruct((BATCH, 1), jnp.float32),
    )(uz, vz, bsum, w1, b1, w2)


def kernel(x, W, H, lin1_w, lin1_b, lin2_w, user_bias, item_bias):
    uidx = x[:, 0]
    iidx = x[:, 1]
    ub16 = user_bias.reshape(-1, _L)
    ib16 = item_bias.reshape(-1, _L)
    uz, vz, bsum = _make_sc_gather()(uidx, iidx, W, H, ub16, ib16)
    return _mlp(uz, vz, bsum.reshape(BATCH, 1), lin1_w,
                lin1_b.reshape(1, EMB_K), lin2_w)
